# baseline (speedup 1.0000x reference)
import numpy as np
import jax
import jax.numpy as jnp
from jax.experimental import pallas as pl

_B, _T, _N = 4096, 336, 21
_D, _M, _P, _K = 64, 5000, 96, 5
_BLK = 256

_S_NP = np.repeat(np.eye(_T, dtype=np.float32), _N, axis=0)


def _split(x):
    hi = x.astype(jnp.bfloat16)
    lo = (x - hi.astype(jnp.float32)).astype(jnp.bfloat16)
    return hi, lo


def _body(x_ref, s_ref, w_ref, b_ref, g_ref, be_ref, mk_ref, mv_ref, thr_ref,
          hist_ref, valid_ref):
    f32 = jnp.float32
    x = x_ref[...]
    xh, xl = _split(x)
    S = s_ref[...]
    q = (jnp.dot(xh, S, preferred_element_type=f32) +
         jnp.dot(xl, S, preferred_element_type=f32)) * (1.0 / _N)

    mean_val = jnp.mean(q, axis=1, keepdims=True)
    qc = q - mean_val
    var = jnp.sum(qc * qc, axis=1, keepdims=True) * (1.0 / (_T - 1))
    std_val = jnp.maximum(jnp.sqrt(var), 1e-6)
    max_val = jnp.max(q, axis=1, keepdims=True)
    min_val = jnp.min(q, axis=1, keepdims=True)
    trend_val = q[:, _T - 1:_T] - q[:, 0:1]

    W = w_ref[...]
    h = (mean_val * W[0:1, :] + std_val * W[1:2, :] + max_val * W[2:3, :]
         + min_val * W[3:4, :] + trend_val * W[4:5, :] + b_ref[...])

    mu = jnp.mean(h, axis=1, keepdims=True)
    hc = h - mu
    lvar = jnp.mean(hc * hc, axis=1, keepdims=True)
    h = hc * jax.lax.rsqrt(lvar + 1e-5)
    h = h * g_ref[...] + be_ref[...]

    nrm = jnp.sqrt(jnp.sum(h * h, axis=1, keepdims=True))
    qk = h / jnp.maximum(nrm, 1e-12)

    qh, ql = _split(qk)
    mk = mk_ref[...]
    kh, kl = _split(mk)
    dn = (((1,), (1,)), ((), ()))
    sim = (jax.lax.dot_general(qh, kh, dn, preferred_element_type=f32) +
           jax.lax.dot_general(qh, kl, dn, preferred_element_type=f32) +
           jax.lax.dot_general(ql, kh, dn, preferred_element_type=f32))

    s = sim
    m0 = jnp.max(s, axis=1, keepdims=True)
    mask = s >= m0
    acc = jnp.where(mask, 1.0, 0.0)
    denom = jnp.ones_like(m0)
    for _ in range(_K - 1):
        s = jnp.where(mask, -jnp.inf, s)
        mk_val = jnp.max(s, axis=1, keepdims=True)
        wk = jnp.exp(mk_val - m0)
        mask = s >= mk_val
        acc = jnp.where(mask, wk, acc)
        denom = denom + wk

    ah, al = _split(acc)
    mv = mv_ref[...]
    vh, vl = _split(mv)
    hist = (jnp.dot(ah, vh, preferred_element_type=f32) +
            jnp.dot(ah, vl, preferred_element_type=f32) +
            jnp.dot(al, vh, preferred_element_type=f32))

    validf = (m0 > thr_ref[...]).astype(f32)
    hist_ref[...] = hist * (validf / denom)
    valid_ref[...] = validf


def kernel(x_normed, W, b, gamma, beta, mem_keys, mem_values, threshold_raw,
           has_extreme):
    B, T, N = x_normed.shape
    x2d = x_normed.reshape(B, T * N)
    thr = jnp.clip(jax.nn.sigmoid(threshold_raw)
                   - has_extreme.astype(jnp.float32) * 0.2, 0.1, None)
    thr2d = thr.reshape(B, 1)
    S = jnp.asarray(_S_NP).astype(jnp.bfloat16)

    nb = B // _BLK
    hist, validf = pl.pallas_call(
        _body,
        grid=(nb,),
        in_specs=[
            pl.BlockSpec((_BLK, T * N), lambda i: (i, 0)),
            pl.BlockSpec((T * N, T), lambda i: (0, 0)),
            pl.BlockSpec((5, _D), lambda i: (0, 0)),
            pl.BlockSpec((1, _D), lambda i: (0, 0)),
            pl.BlockSpec((1, _D), lambda i: (0, 0)),
            pl.BlockSpec((1, _D), lambda i: (0, 0)),
            pl.BlockSpec((_M, _D), lambda i: (0, 0)),
            pl.BlockSpec((_M, _P), lambda i: (0, 0)),
            pl.BlockSpec((_BLK, 1), lambda i: (i, 0)),
        ],
        out_specs=[
            pl.BlockSpec((_BLK, _P), lambda i: (i, 0)),
            pl.BlockSpec((_BLK, 1), lambda i: (i, 0)),
        ],
        out_shape=[
            jax.ShapeDtypeStruct((B, _P), jnp.float32),
            jax.ShapeDtypeStruct((B, 1), jnp.float32),
        ],
    )(x2d, S, W, b.reshape(1, _D), gamma.reshape(1, _D), beta.reshape(1, _D),
      mem_keys, mem_values, thr2d)

    return hist, validf.reshape(B) > 0.5


# speedup vs baseline: 1.0218x; 1.0218x over previous
import jax
import jax.numpy as jnp
from jax.experimental import pallas as pl

_B, _T, _N = 4096, 336, 21
_D, _M, _P, _K = 64, 5000, 96, 5
_BLK = 64


def _split(x):
    hi = x.astype(jnp.bfloat16)
    lo = (x - hi.astype(jnp.float32)).astype(jnp.bfloat16)
    return hi, lo


def _body(x_ref, w_ref, b_ref, g_ref, be_ref, mk_ref, mv_ref, thr_ref,
          hist_ref, valid_ref):
    f32 = jnp.float32
    x = x_ref[...]
    xt = jnp.swapaxes(x, 1, 2)
    q = jnp.sum(xt, axis=1) * (1.0 / _N)

    mean_val = jnp.mean(q, axis=1, keepdims=True)
    qc = q - mean_val
    var = jnp.sum(qc * qc, axis=1, keepdims=True) * (1.0 / (_T - 1))
    std_val = jnp.maximum(jnp.sqrt(var), 1e-6)
    max_val = jnp.max(q, axis=1, keepdims=True)
    min_val = jnp.min(q, axis=1, keepdims=True)
    trend_val = q[:, _T - 1:_T] - q[:, 0:1]

    W = w_ref[...]
    h = (mean_val * W[0:1, :] + std_val * W[1:2, :] + max_val * W[2:3, :]
         + min_val * W[3:4, :] + trend_val * W[4:5, :] + b_ref[...])

    mu = jnp.mean(h, axis=1, keepdims=True)
    hc = h - mu
    lvar = jnp.mean(hc * hc, axis=1, keepdims=True)
    h = hc * jax.lax.rsqrt(lvar + 1e-5)
    h = h * g_ref[...] + be_ref[...]

    nrm = jnp.sqrt(jnp.sum(h * h, axis=1, keepdims=True))
    qk = h / jnp.maximum(nrm, 1e-12)

    qh, ql = _split(qk)
    mk = mk_ref[...]
    kh, kl = _split(mk)
    dn = (((1,), (1,)), ((), ()))
    sim = (jax.lax.dot_general(qh, kh, dn, preferred_element_type=f32) +
           jax.lax.dot_general(qh, kl, dn, preferred_element_type=f32) +
           jax.lax.dot_general(ql, kh, dn, preferred_element_type=f32))

    s = sim
    m0 = jnp.max(s, axis=1, keepdims=True)
    mask = s >= m0
    acc = jnp.where(mask, 1.0, 0.0)
    denom = jnp.ones_like(m0)
    for _ in range(_K - 1):
        s = jnp.where(mask, -jnp.inf, s)
        mk_val = jnp.max(s, axis=1, keepdims=True)
        wk = jnp.exp(mk_val - m0)
        mask = s >= mk_val
        acc = jnp.where(mask, wk, acc)
        denom = denom + wk

    ah, al = _split(acc)
    mv = mv_ref[...]
    vh, vl = _split(mv)
    hist = (jnp.dot(ah, vh, preferred_element_type=f32) +
            jnp.dot(ah, vl, preferred_element_type=f32) +
            jnp.dot(al, vh, preferred_element_type=f32))

    validf = (m0 > thr_ref[...]).astype(f32)
    hist_ref[...] = hist * (validf / denom)
    valid_ref[...] = validf


def kernel(x_normed, W, b, gamma, beta, mem_keys, mem_values, threshold_raw,
           has_extreme):
    B, T, N = x_normed.shape
    thr = jnp.clip(jax.nn.sigmoid(threshold_raw)
                   - has_extreme.astype(jnp.float32) * 0.2, 0.1, None)
    thr2d = thr.reshape(B, 1)

    nb = B // _BLK
    hist, validf = pl.pallas_call(
        _body,
        grid=(nb,),
        in_specs=[
            pl.BlockSpec((_BLK, T, N), lambda i: (i, 0, 0)),
            pl.BlockSpec((5, _D), lambda i: (0, 0)),
            pl.BlockSpec((1, _D), lambda i: (0, 0)),
            pl.BlockSpec((1, _D), lambda i: (0, 0)),
            pl.BlockSpec((1, _D), lambda i: (0, 0)),
            pl.BlockSpec((_M, _D), lambda i: (0, 0)),
            pl.BlockSpec((_M, _P), lambda i: (0, 0)),
            pl.BlockSpec((_BLK, 1), lambda i: (i, 0)),
        ],
        out_specs=[
            pl.BlockSpec((_BLK, _P), lambda i: (i, 0)),
            pl.BlockSpec((_BLK, 1), lambda i: (i, 0)),
        ],
        out_shape=[
            jax.ShapeDtypeStruct((B, _P), jnp.float32),
            jax.ShapeDtypeStruct((B, 1), jnp.float32),
        ],
    )(x_normed, W, b.reshape(1, _D), gamma.reshape(1, _D), beta.reshape(1, _D),
      mem_keys, mem_values, thr2d)

    return hist, validf.reshape(B) > 0.5


# speedup vs baseline: 4.9549x; 4.8491x over previous
import jax
import jax.numpy as jnp
from jax.experimental import pallas as pl

_B, _T, _N = 4096, 336, 21
_D, _M, _P, _K = 64, 5000, 96, 5
_BLK = 256


def _split(x):
    hi = x.astype(jnp.bfloat16)
    lo = (x - hi.astype(jnp.float32)).astype(jnp.bfloat16)
    return hi, lo


def _body(x_ref, w_ref, b_ref, g_ref, be_ref, mk_ref, mv_ref, thr_ref,
          hist_ref, valid_ref):
    f32 = jnp.float32
    x = x_ref[...]
    q = jnp.sum(x, axis=0) * (1.0 / _N)

    mean_val = jnp.mean(q, axis=0, keepdims=True)
    qc = q - mean_val
    var = jnp.sum(qc * qc, axis=0, keepdims=True) * (1.0 / (_T - 1))
    std_val = jnp.maximum(jnp.sqrt(var), 1e-6)
    max_val = jnp.max(q, axis=0, keepdims=True)
    min_val = jnp.min(q, axis=0, keepdims=True)
    trend_val = q[_T - 1:_T, :] - q[0:1, :]

    stats = jnp.concatenate(
        [mean_val, std_val, max_val, min_val, trend_val], axis=0)
    st = jnp.swapaxes(stats, 0, 1)

    W = w_ref[...]
    h = (st[:, 0:1] * W[0:1, :] + st[:, 1:2] * W[1:2, :]
         + st[:, 2:3] * W[2:3, :] + st[:, 3:4] * W[3:4, :]
         + st[:, 4:5] * W[4:5, :] + b_ref[...])

    mu = jnp.mean(h, axis=1, keepdims=True)
    hc = h - mu
    lvar = jnp.mean(hc * hc, axis=1, keepdims=True)
    h = hc * jax.lax.rsqrt(lvar + 1e-5)
    h = h * g_ref[...] + be_ref[...]

    nrm = jnp.sqrt(jnp.sum(h * h, axis=1, keepdims=True))
    qk = h / jnp.maximum(nrm, 1e-12)

    qh, ql = _split(qk)
    mk = mk_ref[...]
    kh, kl = _split(mk)
    dn = (((1,), (1,)), ((), ()))
    sim = (jax.lax.dot_general(qh, kh, dn, preferred_element_type=f32) +
           jax.lax.dot_general(qh, kl, dn, preferred_element_type=f32) +
           jax.lax.dot_general(ql, kh, dn, preferred_element_type=f32))

    s = sim
    m0 = jnp.max(s, axis=1, keepdims=True)
    mask = s >= m0
    acc = jnp.where(mask, 1.0, 0.0)
    denom = jnp.ones_like(m0)
    for _ in range(_K - 1):
        s = jnp.where(mask, -jnp.inf, s)
        mk_val = jnp.max(s, axis=1, keepdims=True)
        wk = jnp.exp(mk_val - m0)
        mask = s >= mk_val
        acc = jnp.where(mask, wk, acc)
        denom = denom + wk

    ah, al = _split(acc)
    mv = mv_ref[...]
    vh, vl = _split(mv)
    hist = (jnp.dot(ah, vh, preferred_element_type=f32) +
            jnp.dot(ah, vl, preferred_element_type=f32) +
            jnp.dot(al, vh, preferred_element_type=f32))

    validf = (m0 > thr_ref[...]).astype(f32)
    hist_ref[...] = hist * (validf / denom)
    valid_ref[...] = validf


def kernel(x_normed, W, b, gamma, beta, mem_keys, mem_values, threshold_raw,
           has_extreme):
    B, T, N = x_normed.shape
    xt = jnp.transpose(x_normed, (2, 1, 0))
    thr = jnp.clip(jax.nn.sigmoid(threshold_raw)
                   - has_extreme.astype(jnp.float32) * 0.2, 0.1, None)
    thr2d = thr.reshape(B, 1)

    nb = B // _BLK
    hist, validf = pl.pallas_call(
        _body,
        grid=(nb,),
        in_specs=[
            pl.BlockSpec((N, T, _BLK), lambda i: (0, 0, i)),
            pl.BlockSpec((5, _D), lambda i: (0, 0)),
            pl.BlockSpec((1, _D), lambda i: (0, 0)),
            pl.BlockSpec((1, _D), lambda i: (0, 0)),
            pl.BlockSpec((1, _D), lambda i: (0, 0)),
            pl.BlockSpec((_M, _D), lambda i: (0, 0)),
            pl.BlockSpec((_M, _P), lambda i: (0, 0)),
            pl.BlockSpec((_BLK, 1), lambda i: (i, 0)),
        ],
        out_specs=[
            pl.BlockSpec((_BLK, _P), lambda i: (i, 0)),
            pl.BlockSpec((_BLK, 1), lambda i: (i, 0)),
        ],
        out_shape=[
            jax.ShapeDtypeStruct((B, _P), jnp.float32),
            jax.ShapeDtypeStruct((B, 1), jnp.float32),
        ],
    )(xt, W, b.reshape(1, _D), gamma.reshape(1, _D), beta.reshape(1, _D),
      mem_keys, mem_values, thr2d)

    return hist, validf.reshape(B) > 0.5


# speedup vs baseline: 5.1213x; 1.0336x over previous
import jax
import jax.numpy as jnp
from jax.experimental import pallas as pl

_B, _T, _N = 4096, 336, 21
_D, _M, _P, _K = 64, 5000, 96, 5
_BLK = 256


def _split(x):
    hi = x.astype(jnp.bfloat16)
    lo = (x - hi.astype(jnp.float32)).astype(jnp.bfloat16)
    return hi, lo


def _body(x_ref, w_ref, b_ref, g_ref, be_ref, kh_ref, kl_ref, vh_ref, vl_ref,
          thr_ref, hist_ref, valid_ref):
    f32 = jnp.float32
    x = x_ref[...]
    q = jnp.sum(x, axis=0) * (1.0 / _N)

    mean_val = jnp.mean(q, axis=0, keepdims=True)
    qc = q - mean_val
    var = jnp.sum(qc * qc, axis=0, keepdims=True) * (1.0 / (_T - 1))
    std_val = jnp.maximum(jnp.sqrt(var), 1e-6)
    max_val = jnp.max(q, axis=0, keepdims=True)
    min_val = jnp.min(q, axis=0, keepdims=True)
    trend_val = q[_T - 1:_T, :] - q[0:1, :]

    stats = jnp.concatenate(
        [mean_val, std_val, max_val, min_val, trend_val], axis=0)
    st = jnp.swapaxes(stats, 0, 1)

    W = w_ref[...]
    h = (st[:, 0:1] * W[0:1, :] + st[:, 1:2] * W[1:2, :]
         + st[:, 2:3] * W[2:3, :] + st[:, 3:4] * W[3:4, :]
         + st[:, 4:5] * W[4:5, :] + b_ref[...])

    mu = jnp.mean(h, axis=1, keepdims=True)
    hc = h - mu
    lvar = jnp.mean(hc * hc, axis=1, keepdims=True)
    h = hc * jax.lax.rsqrt(lvar + 1e-5)
    h = h * g_ref[...] + be_ref[...]

    nrm = jnp.sqrt(jnp.sum(h * h, axis=1, keepdims=True))
    qk = h / jnp.maximum(nrm, 1e-12)

    qh, ql = _split(qk)
    kh = kh_ref[...]
    kl = kl_ref[...]
    dn = (((1,), (1,)), ((), ()))
    sim = (jax.lax.dot_general(qh, kh, dn, preferred_element_type=f32) +
           jax.lax.dot_general(qh, kl, dn, preferred_element_type=f32) +
           jax.lax.dot_general(ql, kh, dn, preferred_element_type=f32))

    m0 = jnp.max(sim, axis=1, keepdims=True)
    cur = m0
    denom = jnp.ones_like(m0)
    for _ in range(_K - 1):
        sm = jnp.where(sim < cur, sim, -jnp.inf)
        cur = jnp.max(sm, axis=1, keepdims=True)
        denom = denom + jnp.exp(cur - m0)

    sel = sim >= cur
    e = jnp.where(sel, jnp.exp(sim - m0), 0.0)
    eh = e.astype(jnp.bfloat16)
    el = (e - eh.astype(f32)).astype(jnp.bfloat16)
    vh = vh_ref[...]
    vl = vl_ref[...]
    hist = (jnp.dot(eh, vh, preferred_element_type=f32) +
            jnp.dot(eh, vl, preferred_element_type=f32) +
            jnp.dot(el, vh, preferred_element_type=f32))

    validf = (m0 > thr_ref[...]).astype(f32)
    hist_ref[...] = hist * (validf / denom)
    valid_ref[...] = validf


def kernel(x_normed, W, b, gamma, beta, mem_keys, mem_values, threshold_raw,
           has_extreme):
    B, T, N = x_normed.shape
    xt = jnp.transpose(x_normed, (2, 1, 0))
    thr = jnp.clip(jax.nn.sigmoid(threshold_raw)
                   - has_extreme.astype(jnp.float32) * 0.2, 0.1, None)
    thr2d = thr.reshape(B, 1)
    kh = mem_keys.astype(jnp.bfloat16)
    kl = (mem_keys - kh.astype(jnp.float32)).astype(jnp.bfloat16)
    vh = mem_values.astype(jnp.bfloat16)
    vl = (mem_values - vh.astype(jnp.float32)).astype(jnp.bfloat16)

    nb = B // _BLK
    hist, validf = pl.pallas_call(
        _body,
        grid=(nb,),
        in_specs=[
            pl.BlockSpec((N, T, _BLK), lambda i: (0, 0, i)),
            pl.BlockSpec((5, _D), lambda i: (0, 0)),
            pl.BlockSpec((1, _D), lambda i: (0, 0)),
            pl.BlockSpec((1, _D), lambda i: (0, 0)),
            pl.BlockSpec((1, _D), lambda i: (0, 0)),
            pl.BlockSpec((_M, _D), lambda i: (0, 0)),
            pl.BlockSpec((_M, _D), lambda i: (0, 0)),
            pl.BlockSpec((_M, _P), lambda i: (0, 0)),
            pl.BlockSpec((_M, _P), lambda i: (0, 0)),
            pl.BlockSpec((_BLK, 1), lambda i: (i, 0)),
        ],
        out_specs=[
            pl.BlockSpec((_BLK, _P), lambda i: (i, 0)),
            pl.BlockSpec((_BLK, 1), lambda i: (i, 0)),
        ],
        out_shape=[
            jax.ShapeDtypeStruct((B, _P), jnp.float32),
            jax.ShapeDtypeStruct((B, 1), jnp.float32),
        ],
    )(xt, W, b.reshape(1, _D), gamma.reshape(1, _D), beta.reshape(1, _D),
      kh, kl, vh, vl, thr2d)

    return hist, validf.reshape(B) > 0.5


# speedup vs baseline: 5.7839x; 1.1294x over previous
import jax
import jax.numpy as jnp
from jax.experimental import pallas as pl

_B, _T, _N = 4096, 336, 21
_D, _M, _P, _K = 64, 5000, 96, 5
_BLK = 256


def _split(x):
    hi = x.astype(jnp.bfloat16)
    lo = (x - hi.astype(jnp.float32)).astype(jnp.bfloat16)
    return hi, lo


def _body(x_ref, w_ref, b_ref, g_ref, be_ref, kh_ref, kl_ref, vh_ref, vl_ref,
          thr_ref, hist_ref, valid_ref):
    f32 = jnp.float32
    x = x_ref[...]
    q = jnp.sum(x, axis=0) * (1.0 / _N)

    mean_val = jnp.mean(q, axis=0, keepdims=True)
    qc = q - mean_val
    var = jnp.sum(qc * qc, axis=0, keepdims=True) * (1.0 / (_T - 1))
    std_val = jnp.maximum(jnp.sqrt(var), 1e-6)
    max_val = jnp.max(q, axis=0, keepdims=True)
    min_val = jnp.min(q, axis=0, keepdims=True)
    trend_val = q[_T - 1:_T, :] - q[0:1, :]

    stats = jnp.concatenate(
        [mean_val, std_val, max_val, min_val, trend_val], axis=0)
    st = jnp.swapaxes(stats, 0, 1)

    W = w_ref[...]
    h = (st[:, 0:1] * W[0:1, :] + st[:, 1:2] * W[1:2, :]
         + st[:, 2:3] * W[2:3, :] + st[:, 3:4] * W[3:4, :]
         + st[:, 4:5] * W[4:5, :] + b_ref[...])

    mu = jnp.mean(h, axis=1, keepdims=True)
    hc = h - mu
    lvar = jnp.mean(hc * hc, axis=1, keepdims=True)
    h = hc * jax.lax.rsqrt(lvar + 1e-5)
    h = h * g_ref[...] + be_ref[...]

    nrm = jnp.sqrt(jnp.sum(h * h, axis=1, keepdims=True))
    qk = h / jnp.maximum(nrm, 1e-12)

    qh, ql = _split(qk)
    kh = kh_ref[...]
    kl = kl_ref[...]
    dn = (((1,), (1,)), ((), ()))
    sim = (jax.lax.dot_general(qh, kh, dn, preferred_element_type=f32) +
           jax.lax.dot_general(qh, kl, dn, preferred_element_type=f32) +
           jax.lax.dot_general(ql, kh, dn, preferred_element_type=f32))

    m0 = jnp.max(sim, axis=1, keepdims=True)
    cur = m0
    denom = jnp.ones_like(m0)
    for _ in range(_K - 1):
        sm = jnp.where(sim < cur, sim, -jnp.inf)
        cur = jnp.max(sm, axis=1, keepdims=True)
        denom = denom + jnp.exp(cur - m0)

    sel = sim >= cur
    eh = jnp.where(sel, jnp.exp(sim - m0), 0.0).astype(jnp.bfloat16)
    vh = vh_ref[...]
    vl = vl_ref[...]
    hist = (jnp.dot(eh, vh, preferred_element_type=f32) +
            jnp.dot(eh, vl, preferred_element_type=f32))

    validf = (m0 > thr_ref[...]).astype(f32)
    hist_ref[...] = hist * (validf / denom)
    valid_ref[...] = validf


def kernel(x_normed, W, b, gamma, beta, mem_keys, mem_values, threshold_raw,
           has_extreme):
    B, T, N = x_normed.shape
    xt = jnp.transpose(x_normed, (2, 1, 0))
    thr = jnp.clip(jax.nn.sigmoid(threshold_raw)
                   - has_extreme.astype(jnp.float32) * 0.2, 0.1, None)
    thr2d = thr.reshape(B, 1)
    kh = mem_keys.astype(jnp.bfloat16)
    kl = (mem_keys - kh.astype(jnp.float32)).astype(jnp.bfloat16)
    vh = mem_values.astype(jnp.bfloat16)
    vl = (mem_values - vh.astype(jnp.float32)).astype(jnp.bfloat16)

    nb = B // _BLK
    hist, validf = pl.pallas_call(
        _body,
        grid=(nb,),
        in_specs=[
            pl.BlockSpec((N, T, _BLK), lambda i: (0, 0, i)),
            pl.BlockSpec((5, _D), lambda i: (0, 0)),
            pl.BlockSpec((1, _D), lambda i: (0, 0)),
            pl.BlockSpec((1, _D), lambda i: (0, 0)),
            pl.BlockSpec((1, _D), lambda i: (0, 0)),
            pl.BlockSpec((_M, _D), lambda i: (0, 0)),
            pl.BlockSpec((_M, _D), lambda i: (0, 0)),
            pl.BlockSpec((_M, _P), lambda i: (0, 0)),
            pl.BlockSpec((_M, _P), lambda i: (0, 0)),
            pl.BlockSpec((_BLK, 1), lambda i: (i, 0)),
        ],
        out_specs=[
            pl.BlockSpec((_BLK, _P), lambda i: (i, 0)),
            pl.BlockSpec((_BLK, 1), lambda i: (i, 0)),
        ],
        out_shape=[
            jax.ShapeDtypeStruct((B, _P), jnp.float32),
            jax.ShapeDtypeStruct((B, 1), jnp.float32),
        ],
    )(xt, W, b.reshape(1, _D), gamma.reshape(1, _D), beta.reshape(1, _D),
      kh, kl, vh, vl, thr2d)

    return hist, validf.reshape(B) > 0.5


# speedup vs baseline: 6.1554x; 1.0642x over previous
import jax
import jax.numpy as jnp
from jax.experimental import pallas as pl
from jax.experimental.pallas import tpu as pltpu

_B, _T, _N = 4096, 336, 21
_D, _M, _P, _K = 64, 5000, 96, 5
_BLK = 256
_HB = 256


def _split(x):
    hi = x.astype(jnp.bfloat16)
    lo = (x - hi.astype(jnp.float32)).astype(jnp.bfloat16)
    return hi, lo


def _half(x_ref, w_ref, b_ref, g_ref, be_ref, kh_ref, klh_ref, vh_ref, vl_ref,
          thr_ref, hist_ref, valid_ref, lo):
    f32 = jnp.float32
    x = x_ref[:, :, lo:lo + _HB]
    q = jnp.sum(x, axis=0) * (1.0 / _N)

    mean_val = jnp.mean(q, axis=0, keepdims=True)
    sumsq = jnp.sum(q * q, axis=0, keepdims=True)
    var = (sumsq - _T * mean_val * mean_val) * (1.0 / (_T - 1))
    std_val = jnp.maximum(jnp.sqrt(jnp.maximum(var, 0.0)), 1e-6)
    max_val = jnp.max(q, axis=0, keepdims=True)
    min_val = jnp.min(q, axis=0, keepdims=True)
    trend_val = q[_T - 1:_T, :] - q[0:1, :]

    stats = jnp.concatenate(
        [mean_val, std_val, max_val, min_val, trend_val], axis=0)
    st = jnp.swapaxes(stats, 0, 1)

    W = w_ref[...]
    h = (st[:, 0:1] * W[0:1, :] + st[:, 1:2] * W[1:2, :]
         + st[:, 2:3] * W[2:3, :] + st[:, 3:4] * W[3:4, :]
         + st[:, 4:5] * W[4:5, :] + b_ref[...])

    mu = jnp.mean(h, axis=1, keepdims=True)
    hc = h - mu
    lvar = jnp.mean(hc * hc, axis=1, keepdims=True)
    h = hc * jax.lax.rsqrt(lvar + 1e-5)
    h = h * g_ref[...] + be_ref[...]

    nrm = jnp.sqrt(jnp.sum(h * h, axis=1, keepdims=True))
    qk = h / jnp.maximum(nrm, 1e-12)

    qh, ql = _split(qk)
    kh = kh_ref[...]
    klh = klh_ref[...]
    dn = (((1,), (1,)), ((), ()))
    qhl = jnp.concatenate([qh, ql], axis=1)
    sim = (jax.lax.dot_general(qh, kh, dn, preferred_element_type=f32) +
           jax.lax.dot_general(qhl, klh, dn, preferred_element_type=f32))

    m0 = jnp.max(sim, axis=1, keepdims=True)
    cur = m0
    denom = jnp.ones_like(m0)
    for _ in range(_K - 1):
        sm = jnp.where(sim < cur, sim, -jnp.inf)
        cur = jnp.max(sm, axis=1, keepdims=True)
        denom = denom + jnp.exp(cur - m0)

    sel = sim >= cur
    eh = jnp.where(sel, jnp.exp(sim - m0), 0.0).astype(jnp.bfloat16)
    vh = vh_ref[...]
    vl = vl_ref[...]
    hist = (jnp.dot(eh, vh, preferred_element_type=f32) +
            jnp.dot(eh, vl, preferred_element_type=f32))

    validf = (m0 > thr_ref[lo:lo + _HB, :]).astype(f32)
    hist_ref[lo:lo + _HB, :] = hist * (validf / denom)
    valid_ref[lo:lo + _HB, :] = validf


def _body(x_ref, w_ref, b_ref, g_ref, be_ref, kh_ref, klh_ref, vh_ref, vl_ref,
          thr_ref, hist_ref, valid_ref):
    for lo in range(0, _BLK, _HB):
        _half(x_ref, w_ref, b_ref, g_ref, be_ref, kh_ref, klh_ref, vh_ref,
              vl_ref, thr_ref, hist_ref, valid_ref, lo)


def kernel(x_normed, W, b, gamma, beta, mem_keys, mem_values, threshold_raw,
           has_extreme):
    B, T, N = x_normed.shape
    xt = jnp.transpose(x_normed, (2, 1, 0))
    thr = jnp.clip(jax.nn.sigmoid(threshold_raw)
                   - has_extreme.astype(jnp.float32) * 0.2, 0.1, None)
    thr2d = thr.reshape(B, 1)
    kh = mem_keys.astype(jnp.bfloat16)
    kl = (mem_keys - kh.astype(jnp.float32)).astype(jnp.bfloat16)
    klh = jnp.concatenate([kl, kh], axis=1)
    vh = mem_values.astype(jnp.bfloat16)
    vl = (mem_values - vh.astype(jnp.float32)).astype(jnp.bfloat16)

    nb = B // _BLK
    hist, validf = pl.pallas_call(
        _body,
        grid=(nb,),
        in_specs=[
            pl.BlockSpec((N, T, _BLK), lambda i: (0, 0, i)),
            pl.BlockSpec((5, _D), lambda i: (0, 0)),
            pl.BlockSpec((1, _D), lambda i: (0, 0)),
            pl.BlockSpec((1, _D), lambda i: (0, 0)),
            pl.BlockSpec((1, _D), lambda i: (0, 0)),
            pl.BlockSpec((_M, _D), lambda i: (0, 0)),
            pl.BlockSpec((_M, 2 * _D), lambda i: (0, 0)),
            pl.BlockSpec((_M, _P), lambda i: (0, 0)),
            pl.BlockSpec((_M, _P), lambda i: (0, 0)),
            pl.BlockSpec((_BLK, 1), lambda i: (i, 0)),
        ],
        out_specs=[
            pl.BlockSpec((_BLK, _P), lambda i: (i, 0)),
            pl.BlockSpec((_BLK, 1), lambda i: (i, 0)),
        ],
        out_shape=[
            jax.ShapeDtypeStruct((B, _P), jnp.float32),
            jax.ShapeDtypeStruct((B, 1), jnp.float32),
        ],
        compiler_params=pltpu.CompilerParams(
            dimension_semantics=("parallel",)),
    )(xt, W, b.reshape(1, _D), gamma.reshape(1, _D), beta.reshape(1, _D),
      kh, klh, vh, vl, thr2d)

    return hist, validf.reshape(B) > 0.5


# speedup vs baseline: 6.2938x; 1.0225x over previous
import jax
import jax.numpy as jnp
from jax.experimental import pallas as pl
from jax.experimental.pallas import tpu as pltpu

_B, _T, _N = 4096, 336, 21
_D, _M, _P, _K = 64, 5000, 96, 5
_BLK = 256
_HB = 256


def _split(x):
    hi = x.astype(jnp.bfloat16)
    lo = (x - hi.astype(jnp.float32)).astype(jnp.bfloat16)
    return hi, lo


def _half(x_ref, w_ref, b_ref, g_ref, be_ref, k3_ref, v2_ref,
          thr_ref, hist_ref, valid_ref, lo):
    f32 = jnp.float32
    x = x_ref[:, :, lo:lo + _HB]
    q = jnp.sum(x, axis=0) * (1.0 / _N)

    mean_val = jnp.mean(q, axis=0, keepdims=True)
    sumsq = jnp.sum(q * q, axis=0, keepdims=True)
    var = (sumsq - _T * mean_val * mean_val) * (1.0 / (_T - 1))
    std_val = jnp.maximum(jnp.sqrt(jnp.maximum(var, 0.0)), 1e-6)
    max_val = jnp.max(q, axis=0, keepdims=True)
    min_val = jnp.min(q, axis=0, keepdims=True)
    trend_val = q[_T - 1:_T, :] - q[0:1, :]

    stats = jnp.concatenate(
        [mean_val, std_val, max_val, min_val, trend_val], axis=0)
    st = jnp.swapaxes(stats, 0, 1)

    W = w_ref[...]
    h = (st[:, 0:1] * W[0:1, :] + st[:, 1:2] * W[1:2, :]
         + st[:, 2:3] * W[2:3, :] + st[:, 3:4] * W[3:4, :]
         + st[:, 4:5] * W[4:5, :] + b_ref[...])

    mu = jnp.mean(h, axis=1, keepdims=True)
    hc = h - mu
    lvar = jnp.mean(hc * hc, axis=1, keepdims=True)
    h = hc * jax.lax.rsqrt(lvar + 1e-5)
    h = h * g_ref[...] + be_ref[...]

    nrm = jnp.sqrt(jnp.sum(h * h, axis=1, keepdims=True))
    qk = h / jnp.maximum(nrm, 1e-12)

    qh, ql = _split(qk)
    k3 = k3_ref[...]
    dn = (((1,), (1,)), ((), ()))
    q3 = jnp.concatenate([qh, qh, ql], axis=1)
    sim = jax.lax.dot_general(q3, k3, dn, preferred_element_type=f32)

    m0 = jnp.max(sim, axis=1, keepdims=True)
    cur = m0
    denom = jnp.ones_like(m0)
    for _ in range(_K - 1):
        sm = jnp.where(sim < cur, sim, -jnp.inf)
        cur = jnp.max(sm, axis=1, keepdims=True)
        denom = denom + jnp.exp(cur - m0)

    sel = sim >= cur
    eh = jnp.where(sel, jnp.exp(sim - m0), 0.0).astype(jnp.bfloat16)
    v2 = v2_ref[...]
    hist2 = jnp.dot(eh, v2, preferred_element_type=f32)
    hist = hist2[:, :_P] + hist2[:, _P:]

    validf = (m0 > thr_ref[lo:lo + _HB, :]).astype(f32)
    hist_ref[lo:lo + _HB, :] = hist * (validf / denom)
    valid_ref[lo:lo + _HB, :] = validf


def _body(x_ref, w_ref, b_ref, g_ref, be_ref, k3_ref, v2_ref,
          thr_ref, hist_ref, valid_ref):
    for lo in range(0, _BLK, _HB):
        _half(x_ref, w_ref, b_ref, g_ref, be_ref, k3_ref, v2_ref,
              thr_ref, hist_ref, valid_ref, lo)


def kernel(x_normed, W, b, gamma, beta, mem_keys, mem_values, threshold_raw,
           has_extreme):
    B, T, N = x_normed.shape
    xt = jnp.transpose(x_normed, (2, 1, 0))
    thr = jnp.clip(jax.nn.sigmoid(threshold_raw)
                   - has_extreme.astype(jnp.float32) * 0.2, 0.1, None)
    thr2d = thr.reshape(B, 1)
    kh = mem_keys.astype(jnp.bfloat16)
    kl = (mem_keys - kh.astype(jnp.float32)).astype(jnp.bfloat16)
    k3 = jnp.concatenate([kh, kl, kh], axis=1)
    vh = mem_values.astype(jnp.bfloat16)
    vl = (mem_values - vh.astype(jnp.float32)).astype(jnp.bfloat16)
    v2 = jnp.concatenate([vh, vl], axis=1)

    nb = B // _BLK
    hist, validf = pl.pallas_call(
        _body,
        grid=(nb,),
        in_specs=[
            pl.BlockSpec((N, T, _BLK), lambda i: (0, 0, i)),
            pl.BlockSpec((5, _D), lambda i: (0, 0)),
            pl.BlockSpec((1, _D), lambda i: (0, 0)),
            pl.BlockSpec((1, _D), lambda i: (0, 0)),
            pl.BlockSpec((1, _D), lambda i: (0, 0)),
            pl.BlockSpec((_M, 3 * _D), lambda i: (0, 0)),
            pl.BlockSpec((_M, 2 * _P), lambda i: (0, 0)),
            pl.BlockSpec((_BLK, 1), lambda i: (i, 0)),
        ],
        out_specs=[
            pl.BlockSpec((_BLK, _P), lambda i: (i, 0)),
            pl.BlockSpec((_BLK, 1), lambda i: (i, 0)),
        ],
        out_shape=[
            jax.ShapeDtypeStruct((B, _P), jnp.float32),
            jax.ShapeDtypeStruct((B, 1), jnp.float32),
        ],
        compiler_params=pltpu.CompilerParams(
            dimension_semantics=("parallel",)),
    )(xt, W, b.reshape(1, _D), gamma.reshape(1, _D), beta.reshape(1, _D),
      k3, v2, thr2d)

    return hist, validf.reshape(B) > 0.5
